# baseline (speedup 1.0000x reference)
import functools

import jax
import jax.numpy as jnp
from jax import lax
from jax.experimental import pallas as pl
from jax.experimental.pallas import tpu as pltpu
from jax.experimental.pallas import tpu_sc as plsc

_NC = 2
_NS = 16
_NW = _NC * _NS


def _sc_gather(table, idx_flat):
    B = idx_flat.shape[0]
    D = table.shape[1]
    b_per_w = B // _NW
    mesh = plsc.VectorSubcoreMesh(core_axis_name="c", subcore_axis_name="s")

    @functools.partial(
        pl.kernel,
        mesh=mesh,
        out_type=jax.ShapeDtypeStruct((B, D), table.dtype),
        scratch_types=[
            pltpu.VMEM((b_per_w,), jnp.int32),
            pltpu.VMEM((b_per_w, D), table.dtype),
            pltpu.SemaphoreType.DMA,
        ],
    )
    def gather_kernel(table_hbm, idx_hbm, out_hbm, idx_v, rows_v, sem):
        wid = lax.axis_index("s") * _NC + lax.axis_index("c")
        base = wid * b_per_w
        pltpu.sync_copy(idx_hbm.at[pl.ds(base, b_per_w)], idx_v)
        pltpu.async_copy(table_hbm.at[idx_v], rows_v, sem).wait()
        pltpu.sync_copy(rows_v, out_hbm.at[pl.ds(base, b_per_w)])

    return gather_kernel(table, idx_flat)


def _mm_body(x_ref, w_ref, b_ref, o_ref):
    w = w_ref[...].astype(jnp.bfloat16)
    acc = lax.dot_general(
        w, x_ref[...],
        dimension_numbers=(((1,), (1,)), ((), ())),
        preferred_element_type=jnp.float32,
    )
    o_ref[...] = acc + b_ref[...].T


def _tc_matmul(x_bf16, w, b2d, vblk):
    T, E = x_bf16.shape
    V = w.shape[0]
    return pl.pallas_call(
        _mm_body,
        grid=(pl.cdiv(V, vblk),),
        in_specs=[
            pl.BlockSpec((T, E), lambda j: (0, 0)),
            pl.BlockSpec((vblk, E), lambda j: (j, 0)),
            pl.BlockSpec((1, vblk), lambda j: (0, j)),
        ],
        out_specs=pl.BlockSpec((vblk, T), lambda j: (j, 0)),
        out_shape=jax.ShapeDtypeStruct((V, T), jnp.float32),
        compiler_params=pltpu.CompilerParams(
            dimension_semantics=("arbitrary",),
        ),
    )(x_bf16, w, b2d)


def kernel(idx, token_embedding, lm_head_w, lm_head_b):
    B, T = idx.shape
    V, E = token_embedding.shape
    idx_flat = idx.reshape(B * T).astype(jnp.int32)
    tok_emb = _sc_gather(token_embedding, idx_flat)
    x = tok_emb.astype(jnp.bfloat16)
    logits_t = _tc_matmul(x, lm_head_w, lm_head_b.reshape(1, V), 1920)
    return logits_t.T.reshape(B, T, V)


# speedup vs baseline: 1.0098x; 1.0098x over previous
import functools

import jax
import jax.numpy as jnp
from jax import lax
from jax.experimental import pallas as pl
from jax.experimental.pallas import tpu as pltpu
from jax.experimental.pallas import tpu_sc as plsc

_NC = 2
_NS = 16
_NW = _NC * _NS


def _sc_gather(table, idx_flat):
    B = idx_flat.shape[0]
    D = table.shape[1]
    b_per_w = B // _NW
    mesh = plsc.VectorSubcoreMesh(core_axis_name="c", subcore_axis_name="s")

    @functools.partial(
        pl.kernel,
        mesh=mesh,
        out_type=jax.ShapeDtypeStruct((B, D), table.dtype),
        scratch_types=[
            pltpu.VMEM((b_per_w,), jnp.int32),
            pltpu.VMEM((b_per_w, D), table.dtype),
            pltpu.SemaphoreType.DMA,
        ],
    )
    def gather_kernel(table_hbm, idx_hbm, out_hbm, idx_v, rows_v, sem):
        wid = lax.axis_index("s") * _NC + lax.axis_index("c")
        base = wid * b_per_w
        pltpu.sync_copy(idx_hbm.at[pl.ds(base, b_per_w)], idx_v)
        pltpu.async_copy(table_hbm.at[idx_v], rows_v, sem).wait()
        pltpu.sync_copy(rows_v, out_hbm.at[pl.ds(base, b_per_w)])

    return gather_kernel(table, idx_flat)


def _mm_body(x_ref, w_ref, b_ref, o_ref):
    w = w_ref[...].astype(jnp.bfloat16)
    acc = lax.dot_general(
        w, x_ref[...],
        dimension_numbers=(((1,), (1,)), ((), ())),
        preferred_element_type=jnp.float32,
    )
    o_ref[...] = acc + b_ref[...].T


def _tc_matmul(x_bf16, w, b2d, vblk):
    T, E = x_bf16.shape
    V = w.shape[0]
    return pl.pallas_call(
        _mm_body,
        grid=(pl.cdiv(V, vblk),),
        in_specs=[
            pl.BlockSpec((T, E), lambda j: (0, 0)),
            pl.BlockSpec((vblk, E), lambda j: (j, 0)),
            pl.BlockSpec((1, vblk), lambda j: (0, j)),
        ],
        out_specs=pl.BlockSpec((vblk, T), lambda j: (j, 0)),
        out_shape=jax.ShapeDtypeStruct((V, T), jnp.float32),
        compiler_params=pltpu.CompilerParams(
            dimension_semantics=("arbitrary",),
        ),
    )(x_bf16, w, b2d)


def kernel(idx, token_embedding, lm_head_w, lm_head_b):
    B, T = idx.shape
    V, E = token_embedding.shape
    idx_flat = idx.reshape(B * T).astype(jnp.int32)
    tok_emb = _sc_gather(token_embedding, idx_flat)
    x = tok_emb.astype(jnp.bfloat16)
    logits_t = _tc_matmul(x, lm_head_w, lm_head_b.reshape(1, V), 1792)
    return logits_t.T.reshape(B, T, V)
